# baseline (speedup 1.0000x reference)
import functools

import jax
import jax.numpy as jnp
from jax import lax
from jax.experimental import pallas as pl
from jax.experimental.pallas import tpu as pltpu
from jax.experimental.pallas import tpu_sc as plsc

_TIME_SCALE = 288.0
_DAY_SCALE = 7.0


@functools.lru_cache(maxsize=None)
def _build_sc_lookup(B, N, D):
    info = plsc.get_sparse_core_info()
    NC, NS, L = info.num_cores, info.num_subcores, info.num_lanes
    NW = NC * NS
    assert B % NW == 0 and N % L == 0
    rows_per_w = B // NW
    G = N // L
    CH = 512
    NCHUNK = N // CH

    mesh = plsc.VectorSubcoreMesh(core_axis_name="c", subcore_axis_name="s")

    @functools.partial(
        pl.kernel,
        out_type=(
            jax.ShapeDtypeStruct((B, N, D), jnp.float32),
            jax.ShapeDtypeStruct((B, N, D), jnp.float32),
        ),
        mesh=mesh,
        compiler_params=pltpu.CompilerParams(use_tc_tiling_on_sc=False),
        scratch_types=[
            pltpu.VMEM((N,), jnp.float32),
            pltpu.VMEM((N,), jnp.float32),
            pltpu.VMEM((NCHUNK, CH), jnp.int32),
            pltpu.VMEM((NCHUNK, CH), jnp.int32),
            pltpu.VMEM((N, D), jnp.float32),
            pltpu.VMEM((N, D), jnp.float32),
            pltpu.SemaphoreType.DMA,
        ],
    )
    def k(ch1_hbm, ch2_hbm, ttab_hbm, dtab_hbm, out_t_hbm, out_d_hbm,
          c1_v, c2_v, idx_t_v, idx_d_v, rows_t_v, rows_d_v, sem):
        cid = lax.axis_index("c")
        sid = lax.axis_index("s")
        wid = sid * NC + cid

        def row_body(i, carry):
            b = wid * rows_per_w + i
            pltpu.sync_copy(ch1_hbm.at[b], c1_v)
            pltpu.sync_copy(ch2_hbm.at[b], c2_v)
            for g in range(G):
                v1 = c1_v[pl.ds(g * L, L)]
                v2 = c2_v[pl.ds(g * L, L)]
                ti = (v1 * _TIME_SCALE).astype(jnp.int32)
                di = (v2 * _DAY_SCALE).astype(jnp.int32)
                idx_t_v[(g * L) // CH, pl.ds((g * L) % CH, L)] = ti
                idx_d_v[(g * L) // CH, pl.ds((g * L) % CH, L)] = di
            copies = []
            for j in range(NCHUNK):
                copies.append(pltpu.async_copy(
                    ttab_hbm.at[idx_t_v.at[j]],
                    rows_t_v.at[pl.ds(j * CH, CH)], sem))
                copies.append(pltpu.async_copy(
                    dtab_hbm.at[idx_d_v.at[j]],
                    rows_d_v.at[pl.ds(j * CH, CH)], sem))
            for cpy in copies:
                cpy.wait()
            pltpu.sync_copy(rows_t_v, out_t_hbm.at[b])
            pltpu.sync_copy(rows_d_v, out_d_hbm.at[b])
            return carry

        lax.fori_loop(0, rows_per_w, row_body, 0)

    return k


def kernel(history_data, time_in_day_emb, day_in_week_emb):
    B, T, N, C = history_data.shape
    _, D = time_in_day_emb.shape
    ch1 = history_data[:, -1, :, 1]
    ch2 = history_data[:, -1, :, 2]
    k = _build_sc_lookup(B, N, D)
    return k(ch1, ch2, time_in_day_emb, day_in_week_emb)


# speedup vs baseline: 2.2250x; 2.2250x over previous
import functools

import jax
import jax.numpy as jnp
from jax import lax
from jax.experimental import pallas as pl
from jax.experimental.pallas import tpu as pltpu
from jax.experimental.pallas import tpu_sc as plsc

_TIME_SCALE = 288.0
_DAY_SCALE = 7.0


@functools.lru_cache(maxsize=None)
def _build_sc_lookup(B, N, D, Vt, Vd):
    info = plsc.get_sparse_core_info()
    NC, NS, L = info.num_cores, info.num_subcores, info.num_lanes
    NW = NC * NS
    assert B % NW == 0 and N % L == 0
    rows_per_w = B // NW
    G = N // L

    mesh = plsc.VectorSubcoreMesh(core_axis_name="c", subcore_axis_name="s")

    @functools.partial(
        pl.kernel,
        out_type=(
            jax.ShapeDtypeStruct((B, N * D), jnp.float32),
            jax.ShapeDtypeStruct((B, N * D), jnp.float32),
        ),
        mesh=mesh,
        compiler_params=pltpu.CompilerParams(
            use_tc_tiling_on_sc=False, needs_layout_passes=False),
        scratch_types=[
            pltpu.VMEM((Vt * D,), jnp.float32),
            pltpu.VMEM((Vd * D,), jnp.float32),
            pltpu.VMEM((N,), jnp.float32),
            pltpu.VMEM((N,), jnp.float32),
            pltpu.VMEM((N * D,), jnp.float32),
            pltpu.VMEM((N * D,), jnp.float32),
        ],
    )
    def k(ch1_hbm, ch2_hbm, ttab_hbm, dtab_hbm, out_t_hbm, out_d_hbm,
          ttab_v, dtab_v, c1_v, c2_v, rows_t_v, rows_d_v):
        cid = lax.axis_index("c")
        sid = lax.axis_index("s")
        wid = sid * NC + cid
        lane32 = lax.iota(jnp.int32, L) * D

        pltpu.sync_copy(ttab_hbm, ttab_v)
        pltpu.sync_copy(dtab_hbm, dtab_v)

        def row_body(i, carry):
            b = wid * rows_per_w + i
            pltpu.sync_copy(ch1_hbm.at[b], c1_v)
            pltpu.sync_copy(ch2_hbm.at[b], c2_v)

            def group_body(g, carry2):
                v1 = c1_v[pl.ds(g * L, L)]
                v2 = c2_v[pl.ds(g * L, L)]
                ti = (v1 * _TIME_SCALE).astype(jnp.int32) * D
                di = (v2 * _DAY_SCALE).astype(jnp.int32) * D
                n32 = lane32 + g * (L * D)
                for d in range(D):
                    tv = plsc.load_gather(ttab_v, [ti + d])
                    plsc.store_scatter(rows_t_v, [n32 + d], tv)
                    dv = plsc.load_gather(dtab_v, [di + d])
                    plsc.store_scatter(rows_d_v, [n32 + d], dv)
                return carry2

            lax.fori_loop(0, G, group_body, 0)
            pltpu.sync_copy(rows_t_v, out_t_hbm.at[b])
            pltpu.sync_copy(rows_d_v, out_d_hbm.at[b])
            return carry

        lax.fori_loop(0, rows_per_w, row_body, 0)

    return k


def kernel(history_data, time_in_day_emb, day_in_week_emb):
    B, T, N, C = history_data.shape
    Vt, D = time_in_day_emb.shape
    Vd, _ = day_in_week_emb.shape
    ch1 = history_data[:, -1, :, 1]
    ch2 = history_data[:, -1, :, 2]
    k = _build_sc_lookup(B, N, D, Vt, Vd)
    out_t, out_d = k(ch1, ch2, time_in_day_emb.reshape(-1),
                     day_in_week_emb.reshape(-1))
    return (out_t.reshape(B, N, D), out_d.reshape(B, N, D))


# speedup vs baseline: 3.4791x; 1.5636x over previous
import functools

import jax
import jax.numpy as jnp
from jax import lax
from jax.experimental import pallas as pl
from jax.experimental.pallas import tpu as pltpu
from jax.experimental.pallas import tpu_sc as plsc

_TIME_SCALE = 288.0
_DAY_SCALE = 7.0


@functools.lru_cache(maxsize=None)
def _build_sc_lookup(B, N, D, Vt, Vd):
    info = plsc.get_sparse_core_info()
    NC, NS, L = info.num_cores, info.num_subcores, info.num_lanes
    NW = NC * NS
    assert B % NW == 0 and N % L == 0
    rows_per_w = B // NW
    G = N // L

    mesh = plsc.VectorSubcoreMesh(core_axis_name="c", subcore_axis_name="s")

    @functools.partial(
        pl.kernel,
        out_type=(
            jax.ShapeDtypeStruct((B, N * D), jnp.float32),
            jax.ShapeDtypeStruct((B, N * D), jnp.float32),
        ),
        mesh=mesh,
        compiler_params=pltpu.CompilerParams(
            use_tc_tiling_on_sc=False, needs_layout_passes=False),
        scratch_types=[
            pltpu.VMEM((Vt * D,), jnp.float32),
            pltpu.VMEM((Vd * D,), jnp.float32),
            pltpu.VMEM((N,), jnp.float32),
            pltpu.VMEM((N,), jnp.float32),
            pltpu.VMEM((N * D,), jnp.float32),
            pltpu.VMEM((N * D,), jnp.float32),
        ],
    )
    def k(ch1_hbm, ch2_hbm, ttab_hbm, dtab_hbm, out_t_hbm, out_d_hbm,
          ttab_v, dtab_v, c1_v, c2_v, rows_t_v, rows_d_v):
        cid = lax.axis_index("c")
        sid = lax.axis_index("s")
        wid = sid * NC + cid
        lane32 = lax.iota(jnp.int32, L) * D

        pltpu.sync_copy(ttab_hbm, ttab_v)
        pltpu.sync_copy(dtab_hbm, dtab_v)

        def row_body(i, carry):
            b = wid * rows_per_w + i
            pltpu.sync_copy(ch1_hbm.at[b], c1_v)
            pltpu.sync_copy(ch2_hbm.at[b], c2_v)

            @plsc.parallel_loop(0, G)
            def group_body(g):
                v1 = c1_v[pl.ds(g * L, L)]
                v2 = c2_v[pl.ds(g * L, L)]
                ti = (v1 * _TIME_SCALE).astype(jnp.int32) * D
                di = (v2 * _DAY_SCALE).astype(jnp.int32) * D
                n32 = lane32 + g * (L * D)
                for d in range(D):
                    tv = plsc.load_gather(ttab_v, [ti + d])
                    plsc.store_scatter(rows_t_v, [n32 + d], tv)
                    dv = plsc.load_gather(dtab_v, [di + d])
                    plsc.store_scatter(rows_d_v, [n32 + d], dv)
            pltpu.sync_copy(rows_t_v, out_t_hbm.at[b])
            pltpu.sync_copy(rows_d_v, out_d_hbm.at[b])
            return carry

        lax.fori_loop(0, rows_per_w, row_body, 0)

    return k


def kernel(history_data, time_in_day_emb, day_in_week_emb):
    B, T, N, C = history_data.shape
    Vt, D = time_in_day_emb.shape
    Vd, _ = day_in_week_emb.shape
    ch1 = history_data[:, -1, :, 1]
    ch2 = history_data[:, -1, :, 2]
    k = _build_sc_lookup(B, N, D, Vt, Vd)
    out_t, out_d = k(ch1, ch2, time_in_day_emb.reshape(-1),
                     day_in_week_emb.reshape(-1))
    return (out_t.reshape(B, N, D), out_d.reshape(B, N, D))


# speedup vs baseline: 7.1695x; 2.0607x over previous
import functools

import jax
import jax.numpy as jnp
from jax import lax
from jax.experimental import pallas as pl
from jax.experimental.pallas import tpu as pltpu
from jax.experimental.pallas import tpu_sc as plsc

_TIME_SCALE = 288.0
_DAY_SCALE = 7.0


@functools.lru_cache(maxsize=None)
def _build_sc_lookup(B, N, D, Vt, Vd):
    info = plsc.get_sparse_core_info()
    NC, NS, L = info.num_cores, info.num_subcores, info.num_lanes
    NW = NC * NS
    assert B % NW == 0 and N % L == 0
    rows_per_w = B // NW
    G = N // L

    mesh = plsc.VectorSubcoreMesh(core_axis_name="c", subcore_axis_name="s")

    @functools.partial(
        pl.kernel,
        out_type=(
            jax.ShapeDtypeStruct((B, N * D), jnp.float32),
            jax.ShapeDtypeStruct((B, N * D), jnp.float32),
        ),
        mesh=mesh,
        compiler_params=pltpu.CompilerParams(
            use_tc_tiling_on_sc=False, needs_layout_passes=False),
        scratch_types=[
            pltpu.VMEM((Vt * D,), jnp.float32),
            pltpu.VMEM((Vd * D,), jnp.float32),
            pltpu.VMEM((N,), jnp.float32),
            pltpu.VMEM((N,), jnp.float32),
            pltpu.VMEM((N * D,), jnp.float32),
            pltpu.VMEM((N * D,), jnp.float32),
        ],
    )
    def k(ch1_hbm, ch2_hbm, ttab_hbm, dtab_hbm, out_t_hbm, out_d_hbm,
          ttab_v, dtab_v, c1_v, c2_v, rows_t_v, rows_d_v):
        cid = lax.axis_index("c")
        sid = lax.axis_index("s")
        wid = sid * NC + cid
        lane = lax.iota(jnp.int32, L)
        lane32 = lane * D
        dds = [(lane + d) & (D - 1) for d in range(D)]

        pltpu.sync_copy(ttab_hbm, ttab_v)
        pltpu.sync_copy(dtab_hbm, dtab_v)

        def row_body(i, carry):
            b = wid * rows_per_w + i
            pltpu.sync_copy(ch1_hbm.at[b], c1_v)
            pltpu.sync_copy(ch2_hbm.at[b], c2_v)

            @plsc.parallel_loop(0, G)
            def group_body(g):
                v1 = c1_v[pl.ds(g * L, L)]
                v2 = c2_v[pl.ds(g * L, L)]
                ti = (v1 * _TIME_SCALE).astype(jnp.int32) * D
                di = (v2 * _DAY_SCALE).astype(jnp.int32) * D
                n32 = lane32 + g * (L * D)
                for d in range(D):
                    dd = dds[d]
                    tv = plsc.load_gather(ttab_v, [ti + dd])
                    plsc.store_scatter(rows_t_v, [n32 + dd], tv)
                    dv = plsc.load_gather(dtab_v, [di + dd])
                    plsc.store_scatter(rows_d_v, [n32 + dd], dv)
            pltpu.sync_copy(rows_t_v, out_t_hbm.at[b])
            pltpu.sync_copy(rows_d_v, out_d_hbm.at[b])
            return carry

        lax.fori_loop(0, rows_per_w, row_body, 0)

    return k


def kernel(history_data, time_in_day_emb, day_in_week_emb):
    B, T, N, C = history_data.shape
    Vt, D = time_in_day_emb.shape
    Vd, _ = day_in_week_emb.shape
    ch1 = history_data[:, -1, :, 1]
    ch2 = history_data[:, -1, :, 2]
    k = _build_sc_lookup(B, N, D, Vt, Vd)
    out_t, out_d = k(ch1, ch2, time_in_day_emb.reshape(-1),
                     day_in_week_emb.reshape(-1))
    return (out_t.reshape(B, N, D), out_d.reshape(B, N, D))
